# baseline (speedup 1.0000x reference)
import functools

import jax
import jax.numpy as jnp
import numpy as np
from jax import lax
from jax.experimental import pallas as pl
from jax.experimental.pallas import tpu as pltpu
from jax.experimental.pallas import tpu_sc as plsc

B, C, H, W = 3, 256, 128, 128
HW = H * W
NC, NS, LANES = 2, 16, 16
NW = NC * NS
QPW = HW // NW
CHUNK = 16
NCHUNK = QPW // CHUNK

_mesh = plsc.VectorSubcoreMesh(core_axis_name="c", subcore_axis_name="s")
_sc_params = pltpu.CompilerParams(use_tc_tiling_on_sc=False)


def _floorf(v):
    t = v.astype(jnp.int32)
    tf = t.astype(jnp.float32)
    t = jnp.where(tf > v, t - 1, t)
    return t, t.astype(jnp.float32)


def _corners(ix, iy, rowoff):
    ix = jnp.clip(ix, -4.0, W + 4.0)
    iy = jnp.clip(iy, -4.0, H + 4.0)
    x0i, x0f = _floorf(ix)
    y0i, y0f = _floorf(iy)
    fx = ix - x0f
    fy = iy - y0f
    res = []
    for dx in (0, 1):
        for dy in (0, 1):
            xc = x0i + dx
            yc = y0i + dy
            wx = fx if dx else (1.0 - fx)
            wy = fy if dy else (1.0 - fy)
            valid = (xc >= 0) & (xc <= W - 1) & (yc >= 0) & (yc <= H - 1)
            wgt = jnp.where(valid, wx * wy, jnp.zeros_like(wx))
            xcc = jnp.clip(xc, 0, W - 1)
            ycc = jnp.clip(yc, 0, H - 1)
            res.append((rowoff + ycc * W + xcc, wgt))
    return res


CW = C // 2
_MASKHI = jnp.int32(-65536)

_PERM = np.empty((C,), np.int32)
for _g in range(C // 32):
    for _i in range(16):
        _PERM[32 * _g + _i] = 32 * _g + 2 * _i
        _PERM[32 * _g + 16 + _i] = 32 * _g + 2 * _i + 1
_IPERM = np.argsort(_PERM).astype(np.int32)


def _unpack_words(w):
    va = lax.bitcast_convert_type(lax.shift_left(w, 16), jnp.float32)
    vb = lax.bitcast_convert_type(lax.bitwise_and(w, _MASKHI), jnp.float32)
    return va, vb


def _pack_words(a, b):
    ua = lax.bitcast_convert_type(a, jnp.int32)
    ub = lax.bitcast_convert_type(b, jnp.int32)
    ra = lax.shift_right_logical(
        ua + 0x7FFF + lax.bitwise_and(lax.shift_right_logical(ua, 16), 1), 16)
    rb = lax.bitwise_and(
        ub + 0x7FFF + lax.bitwise_and(lax.shift_right_logical(ub, 16), 1), _MASKHI)
    return lax.bitwise_or(ra, rb)


_GATHER_DN = jax.lax.GatherDimensionNumbers(
    offset_dims=(), collapsed_slice_dims=(0,), start_index_map=(0,))


def _bcast(v, q):
    idx = jnp.full((LANES, 1), q, jnp.int32)
    return lax.gather(v, idx, _GATHER_DN, (1,),
                      mode=lax.GatherScatterMode.PROMISE_IN_BOUNDS)


@functools.partial(
    pl.kernel,
    out_type=(jax.ShapeDtypeStruct((B * HW, C), jnp.float32),
              jax.ShapeDtypeStruct((B * HW, CW), jnp.int32)),
    mesh=_mesh,
    scratch_types=[
        pltpu.VMEM((2, LANES), jnp.float32),
        pltpu.VMEM((4 * CHUNK,), jnp.int32),
        pltpu.VMEM((4 * CHUNK,), jnp.int32),
        pltpu.VMEM((4 * CHUNK, CW), jnp.int32),
        pltpu.VMEM((4 * CHUNK, CW), jnp.int32),
        pltpu.VMEM((CHUNK, C), jnp.float32),
        pltpu.VMEM((CHUNK, CW), jnp.int32),
        pltpu.SemaphoreType.DMA,
        pltpu.SemaphoreType.DMA,
    ],
    compiler_params=_sc_params,
)
def _warp_k(xt_hbm, wxy_hbm, v_hbm, vw_hbm, gvm, idxa, idxb, rowsa, rowsb,
            outb, outbw, sema, semb):
    wid = lax.axis_index("s") * NC + lax.axis_index("c")

    def read_gvm(b, cn):
        qbase = wid * QPW + cn * CHUNK
        pltpu.sync_copy(wxy_hbm.at[b, pl.ds(0, 2), pl.ds(qbase, LANES)], gvm)

    def prep(b, idxv):
        cw = _corners(gvm[0, :], gvm[1, :], b * HW)
        for k, (idx, _) in enumerate(cw):
            idxv[pl.ds(k * LANES, LANES)] = idx
        return tuple(wv for (_, wv) in cw)

    def fire(idxv, rows, sem):
        pltpu.make_async_copy(xt_hbm.at[idxv], rows, sem).start()

    def wait(idxv, rows, sem):
        pltpu.make_async_copy(xt_hbm.at[idxv], rows, sem).wait()

    def accum(rows, wgts, b, cn):
        def q_body(q, inner):
            accs = [jnp.zeros((LANES,), jnp.float32)] * (C // LANES)
            for k in range(4):
                wv = _bcast(wgts[k], q)
                m = k * LANES + q
                for g in range(CW // LANES):
                    wrd = rows[m, pl.ds(g * LANES, LANES)]
                    va, vb = _unpack_words(wrd)
                    accs[2 * g] = accs[2 * g] + wv * va
                    accs[2 * g + 1] = accs[2 * g + 1] + wv * vb
            for j in range(C // LANES):
                outb[q, pl.ds(j * LANES, LANES)] = accs[j]
            for g in range(CW // LANES):
                outbw[q, pl.ds(g * LANES, LANES)] = _pack_words(
                    accs[2 * g], accs[2 * g + 1])
            return inner

        lax.fori_loop(0, CHUNK, q_body, 0)
        base = b * HW + wid * QPW + cn * CHUNK
        pltpu.sync_copy(outb, v_hbm.at[pl.ds(base, CHUNK)])
        pltpu.sync_copy(outbw, vw_hbm.at[pl.ds(base, CHUNK)])

    for b in range(B):
        read_gvm(b, 0)
        wa0 = prep(b, idxa)
        fire(idxa, rowsa, sema)

        def pair_body(i, wa, b=b):
            read_gvm(b, 2 * i + 1)
            wb = prep(b, idxb)
            fire(idxb, rowsb, semb)
            wait(idxa, rowsa, sema)
            accum(rowsa, wa, b, 2 * i)

            @pl.when(i < NCHUNK // 2 - 1)
            def _():
                read_gvm(b, 2 * i + 2)

            wa2 = prep(b, idxa)

            @pl.when(i < NCHUNK // 2 - 1)
            def _():
                fire(idxa, rowsa, sema)

            wait(idxb, rowsb, semb)
            accum(rowsb, wb, b, 2 * i + 1)
            return wa2

        lax.fori_loop(0, NCHUNK // 2, pair_body, wa0)


BQ = 512


def _proj_body(v_ref, wc_ref, bc_ref, p_ref):
    j = pl.program_id(1)
    vblk = v_ref[0]
    ot = lax.dot_general(wc_ref[...], vblk, (((0,), (1,)), ((), ())),
                         preferred_element_type=jnp.float32)
    ot = ot + bc_ref[...][:, 0:1]
    ot8 = ot[0:8]
    att = ot[8:12]
    m = jnp.max(att, axis=0, keepdims=True)
    e = jnp.exp(att - m)
    aw = e / jnp.sum(e, axis=0, keepdims=True)
    qid = j * BQ + lax.broadcasted_iota(jnp.int32, (8, BQ), 1)
    par = lax.broadcasted_iota(jnp.int32, (8, BQ), 0) & 1
    pxf = (qid & (W - 1)).astype(jnp.float32)
    pyf = (qid >> 7).astype(jnp.float32)
    pos = ot8 + jnp.where(par == 0, pxf, pyf)
    p_ref[0, 0:8, :] = pos
    p_ref[0, 8:12, :] = aw
    p_ref[0, 12:16, :] = jnp.zeros((4, BQ), jnp.float32)


_proj = pl.pallas_call(
    _proj_body,
    grid=(B, HW // BQ),
    in_specs=[
        pl.BlockSpec((1, BQ, C), lambda b, j: (b, j, 0)),
        pl.BlockSpec((C, 32), lambda b, j: (0, 0)),
        pl.BlockSpec((32, 128), lambda b, j: (0, 0)),
    ],
    out_specs=pl.BlockSpec((1, 16, BQ), lambda b, j: (b, 0, j)),
    out_shape=jax.ShapeDtypeStruct((B, 16, HW), jnp.float32),
)


@functools.partial(
    pl.kernel,
    out_type=jax.ShapeDtypeStruct((B * HW, C), jnp.float32),
    mesh=_mesh,
    scratch_types=[
        pltpu.VMEM((12, LANES), jnp.float32),
        pltpu.VMEM((8 * CHUNK,), jnp.int32),
        pltpu.VMEM((8 * CHUNK,), jnp.int32),
        pltpu.VMEM((8 * CHUNK, CW), jnp.int32),
        pltpu.VMEM((8 * CHUNK, CW), jnp.int32),
        pltpu.VMEM((CHUNK, C), jnp.float32),
        pltpu.SemaphoreType.DMA,
        pltpu.SemaphoreType.DMA,
    ],
    compiler_params=_sc_params,
)
def _samp_k(v_hbm, p_hbm, o_hbm, pvm, idxa, idxb, bufa, bufb, outb,
            sema, semb):
    wid = lax.axis_index("s") * NC + lax.axis_index("c")

    def read_pvm(b, cn):
        qbase = wid * QPW + cn * CHUNK
        pltpu.sync_copy(p_hbm.at[b, pl.ds(0, 12), pl.ds(qbase, LANES)], pvm)

    def prep(b, p0, idxv):
        ws = []
        for p in (p0, p0 + 1):
            cw = _corners(pvm[2 * p, :], pvm[2 * p + 1, :], b * HW)
            awp = pvm[8 + p, :]
            for k, (idx, wgt) in enumerate(cw):
                r = (p - p0) * 4 + k
                idxv[pl.ds(r * LANES, LANES)] = idx
                ws.append(awp * wgt)
        return tuple(ws)

    def fire(idxv, buf, sem):
        pltpu.make_async_copy(v_hbm.at[idxv], buf, sem).start()

    def wait(idxv, buf, sem):
        pltpu.make_async_copy(v_hbm.at[idxv], buf, sem).wait()

    def accum(buf, ws, first):
        def q_body(q, inner):
            if first:
                accs = [jnp.zeros((LANES,), jnp.float32)] * (C // LANES)
            else:
                accs = [outb[q, pl.ds(j * LANES, LANES)] for j in range(C // LANES)]
            for r in range(8):
                wv = _bcast(ws[r], q)
                m = r * LANES + q
                for g in range(CW // LANES):
                    wrd = buf[m, pl.ds(g * LANES, LANES)]
                    va, vb = _unpack_words(wrd)
                    accs[2 * g] = accs[2 * g] + wv * va
                    accs[2 * g + 1] = accs[2 * g + 1] + wv * vb
            for j in range(C // LANES):
                outb[q, pl.ds(j * LANES, LANES)] = accs[j]
            return inner

        lax.fori_loop(0, CHUNK, q_body, 0)

    for b in range(B):
        read_pvm(b, 0)
        wa0 = prep(b, 0, idxa)
        fire(idxa, bufa, sema)

        def chunk_body(cn, wa, b=b):
            wb = prep(b, 2, idxb)
            fire(idxb, bufb, semb)
            wait(idxa, bufa, sema)
            accum(bufa, wa, first=True)

            @pl.when(cn < NCHUNK - 1)
            def _():
                read_pvm(b, cn + 1)

            wa2 = prep(b, 0, idxa)

            @pl.when(cn < NCHUNK - 1)
            def _():
                fire(idxa, bufa, sema)

            wait(idxb, bufb, semb)
            accum(bufb, wb, first=False)
            pltpu.sync_copy(outb, o_hbm.at[pl.ds(b * HW + wid * QPW + cn * CHUNK, CHUNK)])
            return wa2

        lax.fori_loop(0, NCHUNK, chunk_body, wa0)


def kernel(x, record_len, pairwise_t_matrix, W_off, b_off, W_att, b_att):
    del record_len
    xtb = x.reshape(B, C, HW).transpose(0, 2, 1).astype(jnp.bfloat16)
    xtw = lax.bitcast_convert_type(
        xtb.reshape(B * HW, CW, 2), jnp.int32)

    theta = pairwise_t_matrix[:, 0, 0].astype(jnp.float32)
    xs = jnp.linspace(-1.0, 1.0, W)
    ys = jnp.linspace(-1.0, 1.0, H)
    gy, gx = jnp.meshgrid(ys, xs, indexing='ij')
    base = jnp.stack([gx, gy, jnp.ones_like(gx)], axis=-1)
    grid = jnp.einsum('nij,hwj->nhwi', theta, base)
    wix = (grid[..., 0].reshape(B, HW) + 1.0) * 0.5 * (W - 1)
    wiy = (grid[..., 1].reshape(B, HW) + 1.0) * 0.5 * (H - 1)
    wxy = jnp.stack([wix, wiy], axis=1)

    wc = jnp.concatenate(
        [W_off[:, :8], W_att[:, :4], jnp.zeros((C, 20), jnp.float32)], axis=1)
    wc = wc[_PERM]
    bc = jnp.concatenate(
        [b_off[:8], b_att[:4], jnp.zeros((20,), jnp.float32)])
    bc128 = jnp.broadcast_to(bc[:, None], (32, 128))

    v, vw = _warp_k(xtw, wxy)
    p = _proj(v.reshape(B, HW, C), wc, bc128)
    o = _samp_k(vw, p)
    o = o.reshape(B, HW, C)[:, :, _IPERM]
    return o.transpose(0, 2, 1).reshape(B, C, H, W)


# speedup vs baseline: 1.4317x; 1.4317x over previous
import functools

import jax
import jax.numpy as jnp
import numpy as np
from jax import lax
from jax.experimental import pallas as pl
from jax.experimental.pallas import tpu as pltpu
from jax.experimental.pallas import tpu_sc as plsc

B, C, H, W = 3, 256, 128, 128
HW = H * W
NC, NS, LANES = 2, 16, 16
NW = NC * NS
QPW = HW // NW
CHUNK = 16
NCHUNK = QPW // CHUNK

_mesh = plsc.VectorSubcoreMesh(core_axis_name="c", subcore_axis_name="s")
_sc_params = pltpu.CompilerParams(use_tc_tiling_on_sc=False)


def _floorf(v):
    t = v.astype(jnp.int32)
    tf = t.astype(jnp.float32)
    t = jnp.where(tf > v, t - 1, t)
    return t, t.astype(jnp.float32)


def _corners(ix, iy, rowoff):
    ix = jnp.clip(ix, -4.0, W + 4.0)
    iy = jnp.clip(iy, -4.0, H + 4.0)
    x0i, x0f = _floorf(ix)
    y0i, y0f = _floorf(iy)
    fx = ix - x0f
    fy = iy - y0f
    res = []
    for dx in (0, 1):
        for dy in (0, 1):
            xc = x0i + dx
            yc = y0i + dy
            wx = fx if dx else (1.0 - fx)
            wy = fy if dy else (1.0 - fy)
            valid = (xc >= 0) & (xc <= W - 1) & (yc >= 0) & (yc <= H - 1)
            wgt = jnp.where(valid, wx * wy, jnp.zeros_like(wx))
            xcc = jnp.clip(xc, 0, W - 1)
            ycc = jnp.clip(yc, 0, H - 1)
            res.append((rowoff + ycc * W + xcc, wgt))
    return res


CW = C // 2
_MASKHI = jnp.int32(-65536)


def _unpack_words(w):
    va = lax.bitcast_convert_type(lax.shift_left(w, 16), jnp.float32)
    vb = lax.bitcast_convert_type(lax.bitwise_and(w, _MASKHI), jnp.float32)
    return va, vb


def _pack_words(a, b):
    ua = lax.bitcast_convert_type(a, jnp.int32)
    ub = lax.bitcast_convert_type(b, jnp.int32)
    ra = lax.shift_right_logical(
        ua + 0x7FFF + lax.bitwise_and(lax.shift_right_logical(ua, 16), 1), 16)
    rb = lax.bitwise_and(
        ub + 0x7FFF + lax.bitwise_and(lax.shift_right_logical(ub, 16), 1), _MASKHI)
    return lax.bitwise_or(ra, rb)


_GATHER_DN = jax.lax.GatherDimensionNumbers(
    offset_dims=(), collapsed_slice_dims=(0,), start_index_map=(0,))


def _bcast(v, q):
    idx = jnp.full((LANES, 1), q, jnp.int32)
    return lax.gather(v, idx, _GATHER_DN, (1,),
                      mode=lax.GatherScatterMode.PROMISE_IN_BOUNDS)


@functools.partial(
    pl.kernel,
    out_type=(jax.ShapeDtypeStruct((B * HW, C), jnp.float32),
              jax.ShapeDtypeStruct((B * HW, CW), jnp.int32)),
    mesh=_mesh,
    scratch_types=[
        pltpu.VMEM((2, LANES), jnp.float32),
        pltpu.VMEM((4 * CHUNK,), jnp.int32),
        pltpu.VMEM((4 * CHUNK,), jnp.int32),
        pltpu.VMEM((4 * CHUNK, CW), jnp.int32),
        pltpu.VMEM((4 * CHUNK, CW), jnp.int32),
        pltpu.VMEM((CHUNK, C), jnp.float32),
        pltpu.VMEM((CHUNK, CW), jnp.int32),
        pltpu.SemaphoreType.DMA,
        pltpu.SemaphoreType.DMA,
    ],
    compiler_params=_sc_params,
)
def _warp_k(xt_hbm, wxy_hbm, v_hbm, vw_hbm, gvm, idxa, idxb, rowsa, rowsb,
            outb, outbw, sema, semb):
    wid = lax.axis_index("s") * NC + lax.axis_index("c")

    def read_gvm(b, cn):
        qbase = wid * QPW + cn * CHUNK
        pltpu.sync_copy(wxy_hbm.at[b, pl.ds(0, 2), pl.ds(qbase, LANES)], gvm)

    def prep(b, idxv):
        cw = _corners(gvm[0, :], gvm[1, :], b * HW)
        for k, (idx, _) in enumerate(cw):
            idxv[pl.ds(k * LANES, LANES)] = idx
        return tuple(wv for (_, wv) in cw)

    def fire(idxv, rows, sem):
        pltpu.make_async_copy(xt_hbm.at[idxv], rows, sem).start()

    def wait(idxv, rows, sem):
        pltpu.make_async_copy(xt_hbm.at[idxv], rows, sem).wait()

    def accum(rows, wgts, b, cn):
        def q_body(q, inner):
            accs = [jnp.zeros((LANES,), jnp.float32)] * (C // LANES)
            for k in range(4):
                wv = _bcast(wgts[k], q)
                m = k * LANES + q
                for g in range(CW // LANES):
                    wrd = rows[m, pl.ds(g * LANES, LANES)]
                    va, vb = _unpack_words(wrd)
                    accs[g] = accs[g] + wv * va
                    accs[g + 8] = accs[g + 8] + wv * vb
            for j in range(C // LANES):
                outb[q, pl.ds(j * LANES, LANES)] = accs[j]
            for g in range(CW // LANES):
                outbw[q, pl.ds(g * LANES, LANES)] = _pack_words(
                    accs[g], accs[g + 8])
            return inner

        lax.fori_loop(0, CHUNK, q_body, 0)
        base = b * HW + wid * QPW + cn * CHUNK
        pltpu.sync_copy(outb, v_hbm.at[pl.ds(base, CHUNK)])
        pltpu.sync_copy(outbw, vw_hbm.at[pl.ds(base, CHUNK)])

    for b in range(B):
        read_gvm(b, 0)
        wa0 = prep(b, idxa)
        fire(idxa, rowsa, sema)

        def pair_body(i, wa, b=b):
            read_gvm(b, 2 * i + 1)
            wb = prep(b, idxb)
            fire(idxb, rowsb, semb)
            wait(idxa, rowsa, sema)
            accum(rowsa, wa, b, 2 * i)

            @pl.when(i < NCHUNK // 2 - 1)
            def _():
                read_gvm(b, 2 * i + 2)

            wa2 = prep(b, idxa)

            @pl.when(i < NCHUNK // 2 - 1)
            def _():
                fire(idxa, rowsa, sema)

            wait(idxb, rowsb, semb)
            accum(rowsb, wb, b, 2 * i + 1)
            return wa2

        lax.fori_loop(0, NCHUNK // 2, pair_body, wa0)


BQ = 512


def _proj_body(v_ref, wc_ref, bc_ref, p_ref):
    j = pl.program_id(1)
    vblk = v_ref[0]
    ot = lax.dot_general(wc_ref[...], vblk, (((0,), (1,)), ((), ())),
                         preferred_element_type=jnp.float32)
    ot = ot + bc_ref[...][:, 0:1]
    ot8 = ot[0:8]
    att = ot[8:12]
    m = jnp.max(att, axis=0, keepdims=True)
    e = jnp.exp(att - m)
    aw = e / jnp.sum(e, axis=0, keepdims=True)
    qid = j * BQ + lax.broadcasted_iota(jnp.int32, (8, BQ), 1)
    par = lax.broadcasted_iota(jnp.int32, (8, BQ), 0) & 1
    pxf = (qid & (W - 1)).astype(jnp.float32)
    pyf = (qid >> 7).astype(jnp.float32)
    pos = ot8 + jnp.where(par == 0, pxf, pyf)
    p_ref[0, 0:8, :] = pos
    p_ref[0, 8:12, :] = aw
    p_ref[0, 12:16, :] = jnp.zeros((4, BQ), jnp.float32)


_proj = pl.pallas_call(
    _proj_body,
    grid=(B, HW // BQ),
    in_specs=[
        pl.BlockSpec((1, BQ, C), lambda b, j: (b, j, 0)),
        pl.BlockSpec((C, 32), lambda b, j: (0, 0)),
        pl.BlockSpec((32, 128), lambda b, j: (0, 0)),
    ],
    out_specs=pl.BlockSpec((1, 16, BQ), lambda b, j: (b, 0, j)),
    out_shape=jax.ShapeDtypeStruct((B, 16, HW), jnp.float32),
)


@functools.partial(
    pl.kernel,
    out_type=jax.ShapeDtypeStruct((B * HW, C), jnp.float32),
    mesh=_mesh,
    scratch_types=[
        pltpu.VMEM((12, LANES), jnp.float32),
        pltpu.VMEM((8 * CHUNK,), jnp.int32),
        pltpu.VMEM((8 * CHUNK,), jnp.int32),
        pltpu.VMEM((8 * CHUNK, CW), jnp.int32),
        pltpu.VMEM((8 * CHUNK, CW), jnp.int32),
        pltpu.VMEM((CHUNK, C), jnp.float32),
        pltpu.SemaphoreType.DMA,
        pltpu.SemaphoreType.DMA,
    ],
    compiler_params=_sc_params,
)
def _samp_k(v_hbm, p_hbm, o_hbm, pvm, idxa, idxb, bufa, bufb, outb,
            sema, semb):
    wid = lax.axis_index("s") * NC + lax.axis_index("c")

    def read_pvm(b, cn):
        qbase = wid * QPW + cn * CHUNK
        pltpu.sync_copy(p_hbm.at[b, pl.ds(0, 12), pl.ds(qbase, LANES)], pvm)

    def prep(b, p0, idxv):
        ws = []
        for p in (p0, p0 + 1):
            cw = _corners(pvm[2 * p, :], pvm[2 * p + 1, :], b * HW)
            awp = pvm[8 + p, :]
            for k, (idx, wgt) in enumerate(cw):
                r = (p - p0) * 4 + k
                idxv[pl.ds(r * LANES, LANES)] = idx
                ws.append(awp * wgt)
        return tuple(ws)

    def fire(idxv, buf, sem):
        pltpu.make_async_copy(v_hbm.at[idxv], buf, sem).start()

    def wait(idxv, buf, sem):
        pltpu.make_async_copy(v_hbm.at[idxv], buf, sem).wait()

    def accum(buf, ws, first):
        def q_body(q, inner):
            if first:
                accs = [jnp.zeros((LANES,), jnp.float32)] * (C // LANES)
            else:
                accs = [outb[q, pl.ds(j * LANES, LANES)] for j in range(C // LANES)]
            for r in range(8):
                wv = _bcast(ws[r], q)
                m = r * LANES + q
                for g in range(CW // LANES):
                    wrd = buf[m, pl.ds(g * LANES, LANES)]
                    va, vb = _unpack_words(wrd)
                    accs[g] = accs[g] + wv * va
                    accs[g + 8] = accs[g + 8] + wv * vb
            for j in range(C // LANES):
                outb[q, pl.ds(j * LANES, LANES)] = accs[j]
            return inner

        lax.fori_loop(0, CHUNK, q_body, 0)

    for b in range(B):
        read_pvm(b, 0)
        wa0 = prep(b, 0, idxa)
        fire(idxa, bufa, sema)

        def chunk_body(cn, wa, b=b):
            wb = prep(b, 2, idxb)
            fire(idxb, bufb, semb)
            wait(idxa, bufa, sema)
            accum(bufa, wa, first=True)

            @pl.when(cn < NCHUNK - 1)
            def _():
                read_pvm(b, cn + 1)

            wa2 = prep(b, 0, idxa)

            @pl.when(cn < NCHUNK - 1)
            def _():
                fire(idxa, bufa, sema)

            wait(idxb, bufb, semb)
            accum(bufb, wb, first=False)
            pltpu.sync_copy(outb, o_hbm.at[pl.ds(b * HW + wid * QPW + cn * CHUNK, CHUNK)])
            return wa2

        lax.fori_loop(0, NCHUNK, chunk_body, wa0)


def kernel(x, record_len, pairwise_t_matrix, W_off, b_off, W_att, b_att):
    del record_len
    xtb = x.reshape(B, C, HW).transpose(0, 2, 1).astype(jnp.bfloat16)
    xtb = xtb.reshape(B * HW, C)
    xtw = lax.bitcast_convert_type(
        jnp.stack([xtb[:, :CW], xtb[:, CW:]], axis=-1),
        jnp.int32)

    theta = pairwise_t_matrix[:, 0, 0].astype(jnp.float32)
    xs = jnp.linspace(-1.0, 1.0, W)
    ys = jnp.linspace(-1.0, 1.0, H)
    gy, gx = jnp.meshgrid(ys, xs, indexing='ij')
    base = jnp.stack([gx, gy, jnp.ones_like(gx)], axis=-1)
    grid = jnp.einsum('nij,hwj->nhwi', theta, base)
    wix = (grid[..., 0].reshape(B, HW) + 1.0) * 0.5 * (W - 1)
    wiy = (grid[..., 1].reshape(B, HW) + 1.0) * 0.5 * (H - 1)
    wxy = jnp.stack([wix, wiy], axis=1)

    wc = jnp.concatenate(
        [W_off[:, :8], W_att[:, :4], jnp.zeros((C, 20), jnp.float32)], axis=1)
    bc = jnp.concatenate(
        [b_off[:8], b_att[:4], jnp.zeros((20,), jnp.float32)])
    bc128 = jnp.broadcast_to(bc[:, None], (32, 128))

    v, vw = _warp_k(xtw, wxy)
    p = _proj(v.reshape(B, HW, C), wc, bc128)
    o = _samp_k(vw, p)
    return o.reshape(B, HW, C).transpose(0, 2, 1).reshape(B, C, H, W)


# speedup vs baseline: 1.5391x; 1.0750x over previous
import functools

import jax
import jax.numpy as jnp
import numpy as np
from jax import lax
from jax.experimental import pallas as pl
from jax.experimental.pallas import tpu as pltpu
from jax.experimental.pallas import tpu_sc as plsc

B, C, H, W = 3, 256, 128, 128
HW = H * W
NC, NS, LANES = 2, 16, 16
NW = NC * NS
QPW = HW // NW
CHUNK = 16
NCHUNK = QPW // CHUNK

_mesh = plsc.VectorSubcoreMesh(core_axis_name="c", subcore_axis_name="s")
_sc_params = pltpu.CompilerParams(use_tc_tiling_on_sc=False)


def _floorf(v):
    t = v.astype(jnp.int32)
    tf = t.astype(jnp.float32)
    t = jnp.where(tf > v, t - 1, t)
    return t, t.astype(jnp.float32)


def _corners(ix, iy, rowoff):
    ix = jnp.clip(ix, -4.0, W + 4.0)
    iy = jnp.clip(iy, -4.0, H + 4.0)
    x0i, x0f = _floorf(ix)
    y0i, y0f = _floorf(iy)
    fx = ix - x0f
    fy = iy - y0f
    res = []
    for dx in (0, 1):
        for dy in (0, 1):
            xc = x0i + dx
            yc = y0i + dy
            wx = fx if dx else (1.0 - fx)
            wy = fy if dy else (1.0 - fy)
            valid = (xc >= 0) & (xc <= W - 1) & (yc >= 0) & (yc <= H - 1)
            wgt = jnp.where(valid, wx * wy, jnp.zeros_like(wx))
            xcc = jnp.clip(xc, 0, W - 1)
            ycc = jnp.clip(yc, 0, H - 1)
            res.append((rowoff + ycc * W + xcc, wgt))
    return res


CW = C // 2
_MASKHI = jnp.int32(-65536)


def _unpack_words(w):
    va = lax.bitcast_convert_type(lax.shift_left(w, 16), jnp.float32)
    vb = lax.bitcast_convert_type(lax.bitwise_and(w, _MASKHI), jnp.float32)
    return va, vb


def _pack_words(a, b):
    ua = lax.bitcast_convert_type(a, jnp.int32)
    ub = lax.bitcast_convert_type(b, jnp.int32)
    ra = lax.shift_right_logical(
        ua + 0x7FFF + lax.bitwise_and(lax.shift_right_logical(ua, 16), 1), 16)
    rb = lax.bitwise_and(
        ub + 0x7FFF + lax.bitwise_and(lax.shift_right_logical(ub, 16), 1), _MASKHI)
    return lax.bitwise_or(ra, rb)


_GATHER_DN = jax.lax.GatherDimensionNumbers(
    offset_dims=(), collapsed_slice_dims=(0,), start_index_map=(0,))


def _bcast(v, q):
    idx = jnp.full((LANES, 1), q, jnp.int32)
    return lax.gather(v, idx, _GATHER_DN, (1,),
                      mode=lax.GatherScatterMode.PROMISE_IN_BOUNDS)


@functools.partial(
    pl.kernel,
    out_type=(jax.ShapeDtypeStruct((B * HW, C), jnp.float32),
              jax.ShapeDtypeStruct((B * HW, CW), jnp.int32)),
    mesh=_mesh,
    scratch_types=[
        pltpu.VMEM((2, QPW), jnp.float32),
        pltpu.VMEM((4 * CHUNK,), jnp.int32),
        pltpu.VMEM((4 * CHUNK,), jnp.int32),
        pltpu.VMEM((4 * CHUNK, CW), jnp.int32),
        pltpu.VMEM((4 * CHUNK, CW), jnp.int32),
        pltpu.VMEM((CHUNK, C), jnp.float32),
        pltpu.VMEM((CHUNK, CW), jnp.int32),
        pltpu.SemaphoreType.DMA,
        pltpu.SemaphoreType.DMA,
    ],
    compiler_params=_sc_params,
)
def _warp_k(xt_hbm, wxy_hbm, v_hbm, vw_hbm, gvm, idxa, idxb, rowsa, rowsb,
            outb, outbw, sema, semb):
    wid = lax.axis_index("s") * NC + lax.axis_index("c")

    def read_gvm(b):
        pltpu.sync_copy(
            wxy_hbm.at[b, pl.ds(0, 2), pl.ds(wid * QPW, QPW)], gvm)

    def prep(b, idxv, cn):
        off = jnp.minimum(cn, NCHUNK - 1) * CHUNK
        cw = _corners(gvm[0, pl.ds(off, LANES)], gvm[1, pl.ds(off, LANES)],
                      b * HW)
        for k, (idx, _) in enumerate(cw):
            idxv[pl.ds(k * LANES, LANES)] = idx
        return tuple(wv for (_, wv) in cw)

    def fire(idxv, rows, sem):
        pltpu.make_async_copy(xt_hbm.at[idxv], rows, sem).start()

    def wait(idxv, rows, sem):
        pltpu.make_async_copy(xt_hbm.at[idxv], rows, sem).wait()

    def accum(rows, wgts, b, cn):
        def q_body(q, inner):
            accs = [jnp.zeros((LANES,), jnp.float32)] * (C // LANES)
            for k in range(4):
                wv = _bcast(wgts[k], q)
                m = k * LANES + q
                for g in range(CW // LANES):
                    wrd = rows[m, pl.ds(g * LANES, LANES)]
                    va, vb = _unpack_words(wrd)
                    accs[g] = accs[g] + wv * va
                    accs[g + 8] = accs[g + 8] + wv * vb
            for j in range(C // LANES):
                outb[q, pl.ds(j * LANES, LANES)] = accs[j]
            for g in range(CW // LANES):
                outbw[q, pl.ds(g * LANES, LANES)] = _pack_words(
                    accs[g], accs[g + 8])
            return inner

        lax.fori_loop(0, CHUNK, q_body, 0)
        base = b * HW + wid * QPW + cn * CHUNK
        pltpu.sync_copy(outb, v_hbm.at[pl.ds(base, CHUNK)])
        pltpu.sync_copy(outbw, vw_hbm.at[pl.ds(base, CHUNK)])

    for b in range(B):
        read_gvm(b)
        wa0 = prep(b, idxa, 0)
        fire(idxa, rowsa, sema)

        def pair_body(i, wa, b=b):
            wb = prep(b, idxb, 2 * i + 1)
            fire(idxb, rowsb, semb)
            wait(idxa, rowsa, sema)
            accum(rowsa, wa, b, 2 * i)
            wa2 = prep(b, idxa, 2 * i + 2)

            @pl.when(i < NCHUNK // 2 - 1)
            def _():
                fire(idxa, rowsa, sema)

            wait(idxb, rowsb, semb)
            accum(rowsb, wb, b, 2 * i + 1)
            return wa2

        lax.fori_loop(0, NCHUNK // 2, pair_body, wa0)


BQ = 512


def _proj_body(v_ref, wc_ref, bc_ref, p_ref):
    j = pl.program_id(1)
    vblk = v_ref[0]
    ot = lax.dot_general(wc_ref[...], vblk, (((0,), (1,)), ((), ())),
                         preferred_element_type=jnp.float32)
    ot = ot + bc_ref[...][:, 0:1]
    ot8 = ot[0:8]
    att = ot[8:12]
    m = jnp.max(att, axis=0, keepdims=True)
    e = jnp.exp(att - m)
    aw = e / jnp.sum(e, axis=0, keepdims=True)
    qid = j * BQ + lax.broadcasted_iota(jnp.int32, (8, BQ), 1)
    par = lax.broadcasted_iota(jnp.int32, (8, BQ), 0) & 1
    pxf = (qid & (W - 1)).astype(jnp.float32)
    pyf = (qid >> 7).astype(jnp.float32)
    pos = ot8 + jnp.where(par == 0, pxf, pyf)
    p_ref[0, 0:8, :] = pos
    p_ref[0, 8:12, :] = aw
    p_ref[0, 12:16, :] = jnp.zeros((4, BQ), jnp.float32)


_proj = pl.pallas_call(
    _proj_body,
    grid=(B, HW // BQ),
    in_specs=[
        pl.BlockSpec((1, BQ, C), lambda b, j: (b, j, 0)),
        pl.BlockSpec((C, 32), lambda b, j: (0, 0)),
        pl.BlockSpec((32, 128), lambda b, j: (0, 0)),
    ],
    out_specs=pl.BlockSpec((1, 16, BQ), lambda b, j: (b, 0, j)),
    out_shape=jax.ShapeDtypeStruct((B, 16, HW), jnp.float32),
)


@functools.partial(
    pl.kernel,
    out_type=jax.ShapeDtypeStruct((B * HW, C), jnp.float32),
    mesh=_mesh,
    scratch_types=[
        pltpu.VMEM((12, QPW), jnp.float32),
        pltpu.VMEM((8 * CHUNK,), jnp.int32),
        pltpu.VMEM((8 * CHUNK,), jnp.int32),
        pltpu.VMEM((8 * CHUNK, CW), jnp.int32),
        pltpu.VMEM((8 * CHUNK, CW), jnp.int32),
        pltpu.VMEM((CHUNK, C), jnp.float32),
        pltpu.SemaphoreType.DMA,
        pltpu.SemaphoreType.DMA,
    ],
    compiler_params=_sc_params,
)
def _samp_k(v_hbm, p_hbm, o_hbm, pvm, idxa, idxb, bufa, bufb, outb,
            sema, semb):
    wid = lax.axis_index("s") * NC + lax.axis_index("c")

    def read_pvm(b):
        pltpu.sync_copy(
            p_hbm.at[b, pl.ds(0, 12), pl.ds(wid * QPW, QPW)], pvm)

    def prep(b, p0, idxv, cn):
        off = jnp.minimum(cn, NCHUNK - 1) * CHUNK
        ws = []
        for p in (p0, p0 + 1):
            cw = _corners(pvm[2 * p, pl.ds(off, LANES)],
                          pvm[2 * p + 1, pl.ds(off, LANES)], b * HW)
            awp = pvm[8 + p, pl.ds(off, LANES)]
            for k, (idx, wgt) in enumerate(cw):
                r = (p - p0) * 4 + k
                idxv[pl.ds(r * LANES, LANES)] = idx
                ws.append(awp * wgt)
        return tuple(ws)

    def fire(idxv, buf, sem):
        pltpu.make_async_copy(v_hbm.at[idxv], buf, sem).start()

    def wait(idxv, buf, sem):
        pltpu.make_async_copy(v_hbm.at[idxv], buf, sem).wait()

    def accum(buf, ws, first):
        def q_body(q, inner):
            if first:
                accs = [jnp.zeros((LANES,), jnp.float32)] * (C // LANES)
            else:
                accs = [outb[q, pl.ds(j * LANES, LANES)] for j in range(C // LANES)]
            for r in range(8):
                wv = _bcast(ws[r], q)
                m = r * LANES + q
                for g in range(CW // LANES):
                    wrd = buf[m, pl.ds(g * LANES, LANES)]
                    va, vb = _unpack_words(wrd)
                    accs[g] = accs[g] + wv * va
                    accs[g + 8] = accs[g + 8] + wv * vb
            for j in range(C // LANES):
                outb[q, pl.ds(j * LANES, LANES)] = accs[j]
            return inner

        lax.fori_loop(0, CHUNK, q_body, 0)

    for b in range(B):
        read_pvm(b)
        wa0 = prep(b, 0, idxa, 0)
        fire(idxa, bufa, sema)

        def chunk_body(cn, wa, b=b):
            wb = prep(b, 2, idxb, cn)
            fire(idxb, bufb, semb)
            wait(idxa, bufa, sema)
            accum(bufa, wa, first=True)
            wa2 = prep(b, 0, idxa, cn + 1)

            @pl.when(cn < NCHUNK - 1)
            def _():
                fire(idxa, bufa, sema)

            wait(idxb, bufb, semb)
            accum(bufb, wb, first=False)
            pltpu.sync_copy(outb, o_hbm.at[pl.ds(b * HW + wid * QPW + cn * CHUNK, CHUNK)])
            return wa2

        lax.fori_loop(0, NCHUNK, chunk_body, wa0)


def kernel(x, record_len, pairwise_t_matrix, W_off, b_off, W_att, b_att):
    del record_len
    xtb = x.reshape(B, C, HW).transpose(0, 2, 1).astype(jnp.bfloat16)
    xtb = xtb.reshape(B * HW, C)
    xtw = lax.bitcast_convert_type(
        jnp.stack([xtb[:, :CW], xtb[:, CW:]], axis=-1),
        jnp.int32)

    theta = pairwise_t_matrix[:, 0, 0].astype(jnp.float32)
    xs = jnp.linspace(-1.0, 1.0, W)
    ys = jnp.linspace(-1.0, 1.0, H)
    gy, gx = jnp.meshgrid(ys, xs, indexing='ij')
    base = jnp.stack([gx, gy, jnp.ones_like(gx)], axis=-1)
    grid = jnp.einsum('nij,hwj->nhwi', theta, base)
    wix = (grid[..., 0].reshape(B, HW) + 1.0) * 0.5 * (W - 1)
    wiy = (grid[..., 1].reshape(B, HW) + 1.0) * 0.5 * (H - 1)
    wxy = jnp.stack([wix, wiy], axis=1)

    wc = jnp.concatenate(
        [W_off[:, :8], W_att[:, :4], jnp.zeros((C, 20), jnp.float32)], axis=1)
    bc = jnp.concatenate(
        [b_off[:8], b_att[:4], jnp.zeros((20,), jnp.float32)])
    bc128 = jnp.broadcast_to(bc[:, None], (32, 128))

    v, vw = _warp_k(xtw, wxy)
    p = _proj(v.reshape(B, HW, C), wc, bc128)
    o = _samp_k(vw, p)
    return o.reshape(B, HW, C).transpose(0, 2, 1).reshape(B, C, H, W)


# speedup vs baseline: 1.7128x; 1.1129x over previous
import functools

import jax
import jax.numpy as jnp
import numpy as np
from jax import lax
from jax.experimental import pallas as pl
from jax.experimental.pallas import tpu as pltpu
from jax.experimental.pallas import tpu_sc as plsc

B, C, H, W = 3, 256, 128, 128
HW = H * W
NC, NS, LANES = 2, 16, 16
NW = NC * NS
QPW = HW // NW
CHUNK = 16
NCHUNK = QPW // CHUNK

_mesh = plsc.VectorSubcoreMesh(core_axis_name="c", subcore_axis_name="s")
_sc_params = pltpu.CompilerParams(use_tc_tiling_on_sc=False)


def _floorf(v):
    t = v.astype(jnp.int32)
    tf = t.astype(jnp.float32)
    t = jnp.where(tf > v, t - 1, t)
    return t, t.astype(jnp.float32)


def _corners(ix, iy, rowoff):
    ix = jnp.clip(ix, -4.0, W + 4.0)
    iy = jnp.clip(iy, -4.0, H + 4.0)
    x0i, x0f = _floorf(ix)
    y0i, y0f = _floorf(iy)
    fx = ix - x0f
    fy = iy - y0f
    res = []
    for dx in (0, 1):
        for dy in (0, 1):
            xc = x0i + dx
            yc = y0i + dy
            wx = fx if dx else (1.0 - fx)
            wy = fy if dy else (1.0 - fy)
            valid = (xc >= 0) & (xc <= W - 1) & (yc >= 0) & (yc <= H - 1)
            wgt = jnp.where(valid, wx * wy, jnp.zeros_like(wx))
            xcc = jnp.clip(xc, 0, W - 1)
            ycc = jnp.clip(yc, 0, H - 1)
            res.append((rowoff + ycc * W + xcc, wgt))
    return res


CW = C // 2
_MASKHI = np.int32(-65536)


def _unpack_words(w):
    va = lax.bitcast_convert_type(lax.shift_left(w, 16), jnp.float32)
    vb = lax.bitcast_convert_type(lax.bitwise_and(w, _MASKHI), jnp.float32)
    return va, vb


def _pack_words(a, b):
    ua = lax.bitcast_convert_type(a, jnp.int32)
    ub = lax.bitcast_convert_type(b, jnp.int32)
    ra = lax.shift_right_logical(
        ua + 0x7FFF + lax.bitwise_and(lax.shift_right_logical(ua, 16), 1), 16)
    rb = lax.bitwise_and(
        ub + 0x7FFF + lax.bitwise_and(lax.shift_right_logical(ub, 16), 1), _MASKHI)
    return lax.bitwise_or(ra, rb)


_GATHER_DN = jax.lax.GatherDimensionNumbers(
    offset_dims=(), collapsed_slice_dims=(0,), start_index_map=(0,))


def _bcast(v, q):
    idx = jnp.full((LANES, 1), q, jnp.int32)
    return lax.gather(v, idx, _GATHER_DN, (1,),
                      mode=lax.GatherScatterMode.PROMISE_IN_BOUNDS)


@functools.partial(
    pl.kernel,
    out_type=jax.ShapeDtypeStruct((B * HW, CW), jnp.int32),
    mesh=_mesh,
    scratch_types=[
        pltpu.VMEM((2, QPW), jnp.float32),
        pltpu.VMEM((4 * CHUNK,), jnp.int32),
        pltpu.VMEM((4 * CHUNK,), jnp.int32),
        pltpu.VMEM((4 * CHUNK, CW), jnp.int32),
        pltpu.VMEM((4 * CHUNK, CW), jnp.int32),
        pltpu.VMEM((CHUNK, CW), jnp.int32),
        pltpu.SemaphoreType.DMA,
        pltpu.SemaphoreType.DMA,
    ],
    compiler_params=_sc_params,
)
def _warp_k(xt_hbm, wxy_hbm, vw_hbm, gvm, idxa, idxb, rowsa, rowsb,
            outbw, sema, semb):
    wid = lax.axis_index("s") * NC + lax.axis_index("c")

    def read_gvm(b):
        pltpu.sync_copy(
            wxy_hbm.at[b, pl.ds(0, 2), pl.ds(wid * QPW, QPW)], gvm)

    def prep(b, idxv, cn):
        off = jnp.minimum(cn, NCHUNK - 1) * CHUNK
        cw = _corners(gvm[0, pl.ds(off, LANES)], gvm[1, pl.ds(off, LANES)],
                      b * HW)
        for k, (idx, _) in enumerate(cw):
            idxv[pl.ds(k * LANES, LANES)] = idx
        return tuple(wv for (_, wv) in cw)

    def fire(idxv, rows, sem):
        pltpu.make_async_copy(xt_hbm.at[idxv], rows, sem).start()

    def wait(idxv, rows, sem):
        pltpu.make_async_copy(xt_hbm.at[idxv], rows, sem).wait()

    def accum(rows, wgts, b, cn):
        def q_body(q, inner):
            accs = [jnp.zeros((LANES,), jnp.float32)] * (C // LANES)
            for k in range(4):
                wv = _bcast(wgts[k], q)
                m = k * LANES + q
                for g in range(CW // LANES):
                    wrd = rows[m, pl.ds(g * LANES, LANES)]
                    va, vb = _unpack_words(wrd)
                    accs[g] = accs[g] + wv * va
                    accs[g + 8] = accs[g + 8] + wv * vb
            for g in range(CW // LANES):
                outbw[q, pl.ds(g * LANES, LANES)] = _pack_words(
                    accs[g], accs[g + 8])
            return inner

        lax.fori_loop(0, CHUNK, q_body, 0)
        base = b * HW + wid * QPW + cn * CHUNK
        pltpu.sync_copy(outbw, vw_hbm.at[pl.ds(base, CHUNK)])

    for b in range(B):
        read_gvm(b)
        wa0 = prep(b, idxa, 0)
        fire(idxa, rowsa, sema)

        def pair_body(i, wa, b=b):
            wb = prep(b, idxb, 2 * i + 1)
            fire(idxb, rowsb, semb)
            wait(idxa, rowsa, sema)
            accum(rowsa, wa, b, 2 * i)
            wa2 = prep(b, idxa, 2 * i + 2)

            @pl.when(i < NCHUNK // 2 - 1)
            def _():
                fire(idxa, rowsa, sema)

            wait(idxb, rowsb, semb)
            accum(rowsb, wb, b, 2 * i + 1)
            return wa2

        lax.fori_loop(0, NCHUNK // 2, pair_body, wa0)


BQ = 512


def _proj_body(vw_ref, wc_ref, bc_ref, p_ref):
    j = pl.program_id(1)
    w = vw_ref[0]
    vlow = lax.bitcast_convert_type(lax.shift_left(w, 16), jnp.float32)
    vhigh = lax.bitcast_convert_type(lax.bitwise_and(w, _MASKHI), jnp.float32)
    ot = (lax.dot_general(wc_ref[0:CW], vlow, (((0,), (1,)), ((), ())),
                          preferred_element_type=jnp.float32)
          + lax.dot_general(wc_ref[CW:C], vhigh, (((0,), (1,)), ((), ())),
                            preferred_element_type=jnp.float32))
    ot = ot + bc_ref[...][:, 0:1]
    ot8 = ot[0:8]
    att = ot[8:12]
    m = jnp.max(att, axis=0, keepdims=True)
    e = jnp.exp(att - m)
    aw = e / jnp.sum(e, axis=0, keepdims=True)
    qid = j * BQ + lax.broadcasted_iota(jnp.int32, (8, BQ), 1)
    par = lax.broadcasted_iota(jnp.int32, (8, BQ), 0) & 1
    pxf = (qid & (W - 1)).astype(jnp.float32)
    pyf = (qid >> 7).astype(jnp.float32)
    pos = ot8 + jnp.where(par == 0, pxf, pyf)
    p_ref[0, 0:8, :] = pos
    p_ref[0, 8:12, :] = aw
    p_ref[0, 12:16, :] = jnp.zeros((4, BQ), jnp.float32)


_proj = pl.pallas_call(
    _proj_body,
    grid=(B, HW // BQ),
    in_specs=[
        pl.BlockSpec((1, BQ, CW), lambda b, j: (b, j, 0)),
        pl.BlockSpec((C, 32), lambda b, j: (0, 0)),
        pl.BlockSpec((32, 128), lambda b, j: (0, 0)),
    ],
    out_specs=pl.BlockSpec((1, 16, BQ), lambda b, j: (b, 0, j)),
    out_shape=jax.ShapeDtypeStruct((B, 16, HW), jnp.float32),
)


@functools.partial(
    pl.kernel,
    out_type=jax.ShapeDtypeStruct((B * HW, C), jnp.float32),
    mesh=_mesh,
    scratch_types=[
        pltpu.VMEM((12, QPW), jnp.float32),
        pltpu.VMEM((8 * CHUNK,), jnp.int32),
        pltpu.VMEM((8 * CHUNK,), jnp.int32),
        pltpu.VMEM((8 * CHUNK, CW), jnp.int32),
        pltpu.VMEM((8 * CHUNK, CW), jnp.int32),
        pltpu.VMEM((CHUNK, C), jnp.float32),
        pltpu.SemaphoreType.DMA,
        pltpu.SemaphoreType.DMA,
    ],
    compiler_params=_sc_params,
)
def _samp_k(v_hbm, p_hbm, o_hbm, pvm, idxa, idxb, bufa, bufb, outb,
            sema, semb):
    wid = lax.axis_index("s") * NC + lax.axis_index("c")

    def read_pvm(b):
        pltpu.sync_copy(
            p_hbm.at[b, pl.ds(0, 12), pl.ds(wid * QPW, QPW)], pvm)

    def prep(b, p0, idxv, cn):
        off = jnp.minimum(cn, NCHUNK - 1) * CHUNK
        ws = []
        for p in (p0, p0 + 1):
            cw = _corners(pvm[2 * p, pl.ds(off, LANES)],
                          pvm[2 * p + 1, pl.ds(off, LANES)], b * HW)
            awp = pvm[8 + p, pl.ds(off, LANES)]
            for k, (idx, wgt) in enumerate(cw):
                r = (p - p0) * 4 + k
                idxv[pl.ds(r * LANES, LANES)] = idx
                ws.append(awp * wgt)
        return tuple(ws)

    def fire(idxv, buf, sem):
        pltpu.make_async_copy(v_hbm.at[idxv], buf, sem).start()

    def wait(idxv, buf, sem):
        pltpu.make_async_copy(v_hbm.at[idxv], buf, sem).wait()

    def accum(buf, ws, first):
        def q_body(q, inner):
            if first:
                accs = [jnp.zeros((LANES,), jnp.float32)] * (C // LANES)
            else:
                accs = [outb[q, pl.ds(j * LANES, LANES)] for j in range(C // LANES)]
            for r in range(8):
                wv = _bcast(ws[r], q)
                m = r * LANES + q
                for g in range(CW // LANES):
                    wrd = buf[m, pl.ds(g * LANES, LANES)]
                    va, vb = _unpack_words(wrd)
                    accs[g] = accs[g] + wv * va
                    accs[g + 8] = accs[g + 8] + wv * vb
            for j in range(C // LANES):
                outb[q, pl.ds(j * LANES, LANES)] = accs[j]
            return inner

        lax.fori_loop(0, CHUNK, q_body, 0)

    for b in range(B):
        read_pvm(b)
        wa0 = prep(b, 0, idxa, 0)
        fire(idxa, bufa, sema)

        def chunk_body(cn, wa, b=b):
            wb = prep(b, 2, idxb, cn)
            fire(idxb, bufb, semb)
            wait(idxa, bufa, sema)
            accum(bufa, wa, first=True)
            wa2 = prep(b, 0, idxa, cn + 1)

            @pl.when(cn < NCHUNK - 1)
            def _():
                fire(idxa, bufa, sema)

            wait(idxb, bufb, semb)
            accum(bufb, wb, first=False)
            pltpu.sync_copy(outb, o_hbm.at[pl.ds(b * HW + wid * QPW + cn * CHUNK, CHUNK)])
            return wa2

        lax.fori_loop(0, NCHUNK, chunk_body, wa0)


def kernel(x, record_len, pairwise_t_matrix, W_off, b_off, W_att, b_att):
    del record_len
    xtb = x.reshape(B, C, HW).transpose(0, 2, 1).astype(jnp.bfloat16)
    xtb = xtb.reshape(B * HW, C)
    xtw = lax.bitcast_convert_type(
        jnp.stack([xtb[:, :CW], xtb[:, CW:]], axis=-1),
        jnp.int32)

    theta = pairwise_t_matrix[:, 0, 0].astype(jnp.float32)
    xs = jnp.linspace(-1.0, 1.0, W)
    ys = jnp.linspace(-1.0, 1.0, H)
    gy, gx = jnp.meshgrid(ys, xs, indexing='ij')
    base = jnp.stack([gx, gy, jnp.ones_like(gx)], axis=-1)
    grid = jnp.einsum('nij,hwj->nhwi', theta, base)
    wix = (grid[..., 0].reshape(B, HW) + 1.0) * 0.5 * (W - 1)
    wiy = (grid[..., 1].reshape(B, HW) + 1.0) * 0.5 * (H - 1)
    wxy = jnp.stack([wix, wiy], axis=1)

    wc = jnp.concatenate(
        [W_off[:, :8], W_att[:, :4], jnp.zeros((C, 20), jnp.float32)], axis=1)
    bc = jnp.concatenate(
        [b_off[:8], b_att[:4], jnp.zeros((20,), jnp.float32)])
    bc128 = jnp.broadcast_to(bc[:, None], (32, 128))

    vw = _warp_k(xtw, wxy)
    p = _proj(vw.reshape(B, HW, CW), wc, bc128)
    o = _samp_k(vw, p)
    return o.reshape(B, HW, C).transpose(0, 2, 1).reshape(B, C, H, W)


# speedup vs baseline: 1.8116x; 1.0577x over previous
import functools

import jax
import jax.numpy as jnp
import numpy as np
from jax import lax
from jax.experimental import pallas as pl
from jax.experimental.pallas import tpu as pltpu
from jax.experimental.pallas import tpu_sc as plsc

B, C, H, W = 3, 256, 128, 128
HW = H * W
NC, NS, LANES = 2, 16, 16
NW = NC * NS
QPW = HW // NW
CHUNK = 16
NCHUNK = QPW // CHUNK

_mesh = plsc.VectorSubcoreMesh(core_axis_name="c", subcore_axis_name="s")
_sc_params = pltpu.CompilerParams(use_tc_tiling_on_sc=False)


def _floorf(v):
    t = v.astype(jnp.int32)
    tf = t.astype(jnp.float32)
    t = jnp.where(tf > v, t - 1, t)
    return t, t.astype(jnp.float32)


def _corners(ix, iy, rowoff):
    ix = jnp.clip(ix, -4.0, W + 4.0)
    iy = jnp.clip(iy, -4.0, H + 4.0)
    x0i, x0f = _floorf(ix)
    y0i, y0f = _floorf(iy)
    fx = ix - x0f
    fy = iy - y0f
    res = []
    for dx in (0, 1):
        for dy in (0, 1):
            xc = x0i + dx
            yc = y0i + dy
            wx = fx if dx else (1.0 - fx)
            wy = fy if dy else (1.0 - fy)
            valid = (xc >= 0) & (xc <= W - 1) & (yc >= 0) & (yc <= H - 1)
            wgt = jnp.where(valid, wx * wy, jnp.zeros_like(wx))
            xcc = jnp.clip(xc, 0, W - 1)
            ycc = jnp.clip(yc, 0, H - 1)
            res.append((rowoff + ycc * W + xcc, wgt))
    return res


CW = C // 2
_MASKHI = np.int32(-65536)


def _unpack_words(w):
    va = lax.bitcast_convert_type(lax.shift_left(w, 16), jnp.float32)
    vb = lax.bitcast_convert_type(lax.bitwise_and(w, _MASKHI), jnp.float32)
    return va, vb


def _pack_words(a, b):
    ua = lax.bitcast_convert_type(a, jnp.int32)
    ub = lax.bitcast_convert_type(b, jnp.int32)
    ra = lax.shift_right_logical(
        ua + 0x7FFF + lax.bitwise_and(lax.shift_right_logical(ua, 16), 1), 16)
    rb = lax.bitwise_and(
        ub + 0x7FFF + lax.bitwise_and(lax.shift_right_logical(ub, 16), 1), _MASKHI)
    return lax.bitwise_or(ra, rb)


_GATHER_DN = jax.lax.GatherDimensionNumbers(
    offset_dims=(), collapsed_slice_dims=(0,), start_index_map=(0,))


def _bcast(v, q):
    idx = jnp.full((LANES, 1), q, jnp.int32)
    return lax.gather(v, idx, _GATHER_DN, (1,),
                      mode=lax.GatherScatterMode.PROMISE_IN_BOUNDS)


def _make_warp(b):
    @functools.partial(
        pl.kernel,
        out_type=jax.ShapeDtypeStruct((HW, CW), jnp.int32),
        mesh=_mesh,
        scratch_types=[
            pltpu.VMEM((2, QPW), jnp.float32),
            pltpu.VMEM((4 * CHUNK,), jnp.int32),
            pltpu.VMEM((4 * CHUNK,), jnp.int32),
            pltpu.VMEM((4 * CHUNK, CW), jnp.int32),
            pltpu.VMEM((4 * CHUNK, CW), jnp.int32),
            pltpu.VMEM((CHUNK, CW), jnp.int32),
            pltpu.SemaphoreType.DMA,
            pltpu.SemaphoreType.DMA,
        ],
        compiler_params=_sc_params,
    )
    def warp_b(xt_hbm, wxy_hbm, vw_hbm, gvm, idxa, idxb, rowsa, rowsb,
               outbw, sema, semb):
        wid = lax.axis_index("s") * NC + lax.axis_index("c")

        def prep(idxv, cn):
            off = jnp.minimum(cn, NCHUNK - 1) * CHUNK
            cw = _corners(gvm[0, pl.ds(off, LANES)], gvm[1, pl.ds(off, LANES)],
                          b * HW)
            for k, (idx, _) in enumerate(cw):
                idxv[pl.ds(k * LANES, LANES)] = idx
            return tuple(wv for (_, wv) in cw)

        def fire(idxv, rows, sem):
            pltpu.make_async_copy(xt_hbm.at[idxv], rows, sem).start()

        def wait(idxv, rows, sem):
            pltpu.make_async_copy(xt_hbm.at[idxv], rows, sem).wait()

        def accum(rows, wgts, cn):
            def q_body(q, inner):
                accs = [jnp.zeros((LANES,), jnp.float32)] * (C // LANES)
                for k in range(4):
                    wv = _bcast(wgts[k], q)
                    m = k * LANES + q
                    for g in range(CW // LANES):
                        wrd = rows[m, pl.ds(g * LANES, LANES)]
                        va, vb = _unpack_words(wrd)
                        accs[g] = accs[g] + wv * va
                        accs[g + 8] = accs[g + 8] + wv * vb
                for g in range(CW // LANES):
                    outbw[q, pl.ds(g * LANES, LANES)] = _pack_words(
                        accs[g], accs[g + 8])
                return inner

            lax.fori_loop(0, CHUNK, q_body, 0)
            pltpu.sync_copy(
                outbw, vw_hbm.at[pl.ds(wid * QPW + cn * CHUNK, CHUNK)])

        pltpu.sync_copy(
            wxy_hbm.at[b, pl.ds(0, 2), pl.ds(wid * QPW, QPW)], gvm)
        wa0 = prep(idxa, 0)
        fire(idxa, rowsa, sema)

        def pair_body(i, wa):
            wb = prep(idxb, 2 * i + 1)
            fire(idxb, rowsb, semb)
            wait(idxa, rowsa, sema)
            accum(rowsa, wa, 2 * i)
            wa2 = prep(idxa, 2 * i + 2)

            @pl.when(i < NCHUNK // 2 - 1)
            def _():
                fire(idxa, rowsa, sema)

            wait(idxb, rowsb, semb)
            accum(rowsb, wb, 2 * i + 1)
            return wa2

        lax.fori_loop(0, NCHUNK // 2, pair_body, wa0)

    return warp_b


_WARPS = [_make_warp(b) for b in range(B)]


BQ = 512


def _proj_body(vw_ref, wc_ref, bc_ref, p_ref):
    j = pl.program_id(0)
    w = vw_ref[...]
    vlow = lax.bitcast_convert_type(lax.shift_left(w, 16), jnp.float32)
    vhigh = lax.bitcast_convert_type(lax.bitwise_and(w, _MASKHI), jnp.float32)
    ot = (lax.dot_general(wc_ref[0:CW], vlow, (((0,), (1,)), ((), ())),
                          preferred_element_type=jnp.float32)
          + lax.dot_general(wc_ref[CW:C], vhigh, (((0,), (1,)), ((), ())),
                            preferred_element_type=jnp.float32))
    ot = ot + bc_ref[...][:, 0:1]
    ot8 = ot[0:8]
    att = ot[8:12]
    m = jnp.max(att, axis=0, keepdims=True)
    e = jnp.exp(att - m)
    aw = e / jnp.sum(e, axis=0, keepdims=True)
    qid = j * BQ + lax.broadcasted_iota(jnp.int32, (8, BQ), 1)
    par = lax.broadcasted_iota(jnp.int32, (8, BQ), 0) & 1
    pxf = (qid & (W - 1)).astype(jnp.float32)
    pyf = (qid >> 7).astype(jnp.float32)
    pos = ot8 + jnp.where(par == 0, pxf, pyf)
    p_ref[0:8, :] = pos
    p_ref[8:12, :] = aw
    p_ref[12:16, :] = jnp.zeros((4, BQ), jnp.float32)


_proj = pl.pallas_call(
    _proj_body,
    grid=(HW // BQ,),
    in_specs=[
        pl.BlockSpec((BQ, CW), lambda j: (j, 0)),
        pl.BlockSpec((C, 32), lambda j: (0, 0)),
        pl.BlockSpec((32, 128), lambda j: (0, 0)),
    ],
    out_specs=pl.BlockSpec((16, BQ), lambda j: (0, j)),
    out_shape=jax.ShapeDtypeStruct((16, HW), jnp.float32),
)


def _make_samp():
    @functools.partial(
        pl.kernel,
        out_type=jax.ShapeDtypeStruct((HW, C), jnp.float32),
        mesh=_mesh,
        scratch_types=[
            pltpu.VMEM((12, QPW), jnp.float32),
            pltpu.VMEM((8 * CHUNK,), jnp.int32),
            pltpu.VMEM((8 * CHUNK,), jnp.int32),
            pltpu.VMEM((8 * CHUNK, CW), jnp.int32),
            pltpu.VMEM((8 * CHUNK, CW), jnp.int32),
            pltpu.VMEM((CHUNK, C), jnp.float32),
            pltpu.SemaphoreType.DMA,
            pltpu.SemaphoreType.DMA,
        ],
        compiler_params=_sc_params,
    )
    def samp_b(v_hbm, p_hbm, o_hbm, pvm, idxa, idxb, bufa, bufb, outb,
               sema, semb):
        wid = lax.axis_index("s") * NC + lax.axis_index("c")

        def prep(p0, idxv, cn):
            off = jnp.minimum(cn, NCHUNK - 1) * CHUNK
            ws = []
            for p in (p0, p0 + 1):
                cw = _corners(pvm[2 * p, pl.ds(off, LANES)],
                              pvm[2 * p + 1, pl.ds(off, LANES)], 0)
                awp = pvm[8 + p, pl.ds(off, LANES)]
                for k, (idx, wgt) in enumerate(cw):
                    r = (p - p0) * 4 + k
                    idxv[pl.ds(r * LANES, LANES)] = idx
                    ws.append(awp * wgt)
            return tuple(ws)

        def fire(idxv, buf, sem):
            pltpu.make_async_copy(v_hbm.at[idxv], buf, sem).start()

        def wait(idxv, buf, sem):
            pltpu.make_async_copy(v_hbm.at[idxv], buf, sem).wait()

        def accum(buf, ws, first):
            def q_body(q, inner):
                if first:
                    accs = [jnp.zeros((LANES,), jnp.float32)] * (C // LANES)
                else:
                    accs = [outb[q, pl.ds(j * LANES, LANES)]
                            for j in range(C // LANES)]
                for r in range(8):
                    wv = _bcast(ws[r], q)
                    m = r * LANES + q
                    for g in range(CW // LANES):
                        wrd = buf[m, pl.ds(g * LANES, LANES)]
                        va, vb = _unpack_words(wrd)
                        accs[g] = accs[g] + wv * va
                        accs[g + 8] = accs[g + 8] + wv * vb
                for j in range(C // LANES):
                    outb[q, pl.ds(j * LANES, LANES)] = accs[j]
                return inner

            lax.fori_loop(0, CHUNK, q_body, 0)

        pltpu.sync_copy(
            p_hbm.at[pl.ds(0, 12), pl.ds(wid * QPW, QPW)], pvm)
        wa0 = prep(0, idxa, 0)
        fire(idxa, bufa, sema)

        def chunk_body(cn, wa):
            wb = prep(2, idxb, cn)
            fire(idxb, bufb, semb)
            wait(idxa, bufa, sema)
            accum(bufa, wa, first=True)
            wa2 = prep(0, idxa, cn + 1)

            @pl.when(cn < NCHUNK - 1)
            def _():
                fire(idxa, bufa, sema)

            wait(idxb, bufb, semb)
            accum(bufb, wb, first=False)
            pltpu.sync_copy(outb, o_hbm.at[pl.ds(wid * QPW + cn * CHUNK, CHUNK)])
            return wa2

        lax.fori_loop(0, NCHUNK, chunk_body, wa0)

    return samp_b


_SAMP = _make_samp()


def kernel(x, record_len, pairwise_t_matrix, W_off, b_off, W_att, b_att):
    del record_len
    xtb = x.reshape(B, C, HW).transpose(0, 2, 1).astype(jnp.bfloat16)
    xtb = xtb.reshape(B * HW, C)
    xtw = lax.bitcast_convert_type(
        jnp.stack([xtb[:, :CW], xtb[:, CW:]], axis=-1),
        jnp.int32)

    theta = pairwise_t_matrix[:, 0, 0].astype(jnp.float32)
    xs = jnp.linspace(-1.0, 1.0, W)
    ys = jnp.linspace(-1.0, 1.0, H)
    gy, gx = jnp.meshgrid(ys, xs, indexing='ij')
    base = jnp.stack([gx, gy, jnp.ones_like(gx)], axis=-1)
    grid = jnp.einsum('nij,hwj->nhwi', theta, base)
    wix = (grid[..., 0].reshape(B, HW) + 1.0) * 0.5 * (W - 1)
    wiy = (grid[..., 1].reshape(B, HW) + 1.0) * 0.5 * (H - 1)
    wxy = jnp.stack([wix, wiy], axis=1)

    wc = jnp.concatenate(
        [W_off[:, :8], W_att[:, :4], jnp.zeros((C, 20), jnp.float32)], axis=1)
    bc = jnp.concatenate(
        [b_off[:8], b_att[:4], jnp.zeros((20,), jnp.float32)])
    bc128 = jnp.broadcast_to(bc[:, None], (32, 128))

    outs = []
    for b in range(B):
        vw_b = _WARPS[b](xtw, wxy)
        p_b = _proj(vw_b, wc, bc128)
        o_b = _SAMP(vw_b, p_b)
        outs.append(o_b.T.reshape(1, C, H, W))
    return jnp.concatenate(outs, axis=0)


# speedup vs baseline: 1.8134x; 1.0010x over previous
import functools

import jax
import jax.numpy as jnp
import numpy as np
from jax import lax
from jax.experimental import pallas as pl
from jax.experimental.pallas import tpu as pltpu
from jax.experimental.pallas import tpu_sc as plsc

B, C, H, W = 3, 256, 128, 128
HW = H * W
NC, NS, LANES = 2, 16, 16
NW = NC * NS
QPW = HW // NW
CHUNK = 16
NCHUNK = QPW // CHUNK

_mesh = plsc.VectorSubcoreMesh(core_axis_name="c", subcore_axis_name="s")
_sc_params = pltpu.CompilerParams(use_tc_tiling_on_sc=False)


def _floorf(v):
    t = v.astype(jnp.int32)
    tf = t.astype(jnp.float32)
    t = jnp.where(tf > v, t - 1, t)
    return t, t.astype(jnp.float32)


def _corners(ix, iy, rowoff):
    ix = jnp.clip(ix, -4.0, W + 4.0)
    iy = jnp.clip(iy, -4.0, H + 4.0)
    x0i, x0f = _floorf(ix)
    y0i, y0f = _floorf(iy)
    fx = ix - x0f
    fy = iy - y0f
    res = []
    for dx in (0, 1):
        for dy in (0, 1):
            xc = x0i + dx
            yc = y0i + dy
            wx = fx if dx else (1.0 - fx)
            wy = fy if dy else (1.0 - fy)
            valid = (xc >= 0) & (xc <= W - 1) & (yc >= 0) & (yc <= H - 1)
            wgt = jnp.where(valid, wx * wy, jnp.zeros_like(wx))
            xcc = jnp.clip(xc, 0, W - 1)
            ycc = jnp.clip(yc, 0, H - 1)
            res.append((rowoff + ycc * W + xcc, wgt))
    return res


CW = C // 2
_MASKHI = np.int32(-65536)


def _unpack_words(w):
    va = lax.bitcast_convert_type(lax.shift_left(w, 16), jnp.float32)
    vb = lax.bitcast_convert_type(lax.bitwise_and(w, _MASKHI), jnp.float32)
    return va, vb


def _pack_words(a, b):
    ua = lax.bitcast_convert_type(a, jnp.int32)
    ub = lax.bitcast_convert_type(b, jnp.int32)
    ra = lax.shift_right_logical(
        ua + 0x7FFF + lax.bitwise_and(lax.shift_right_logical(ua, 16), 1), 16)
    rb = lax.bitwise_and(
        ub + 0x7FFF + lax.bitwise_and(lax.shift_right_logical(ub, 16), 1), _MASKHI)
    return lax.bitwise_or(ra, rb)


_GATHER_DN = jax.lax.GatherDimensionNumbers(
    offset_dims=(), collapsed_slice_dims=(0,), start_index_map=(0,))


def _bcast(v, q):
    idx = jnp.full((LANES, 1), q, jnp.int32)
    return lax.gather(v, idx, _GATHER_DN, (1,),
                      mode=lax.GatherScatterMode.PROMISE_IN_BOUNDS)


def _make_warp(b):
    @functools.partial(
        pl.kernel,
        out_type=jax.ShapeDtypeStruct((HW, CW), jnp.int32),
        mesh=_mesh,
        scratch_types=[
            pltpu.VMEM((2, QPW), jnp.float32),
            pltpu.VMEM((4 * CHUNK,), jnp.int32),
            pltpu.VMEM((4 * CHUNK,), jnp.int32),
            pltpu.VMEM((4 * CHUNK, CW), jnp.int32),
            pltpu.VMEM((4 * CHUNK, CW), jnp.int32),
            pltpu.VMEM((CHUNK, CW), jnp.int32),
            pltpu.SemaphoreType.DMA,
            pltpu.SemaphoreType.DMA,
        ],
        compiler_params=_sc_params,
    )
    def warp_b(xt_hbm, wxy_hbm, vw_hbm, gvm, idxa, idxb, rowsa, rowsb,
               outbw, sema, semb):
        wid = lax.axis_index("s") * NC + lax.axis_index("c")

        def prep(idxv, cn):
            off = jnp.minimum(cn, NCHUNK - 1) * CHUNK
            cw = _corners(gvm[0, pl.ds(off, LANES)], gvm[1, pl.ds(off, LANES)],
                          b * HW)
            for k, (idx, _) in enumerate(cw):
                idxv[pl.ds(k * LANES, LANES)] = idx
            return tuple(wv for (_, wv) in cw)

        def fire(idxv, rows, sem):
            pltpu.make_async_copy(xt_hbm.at[idxv], rows, sem).start()

        def wait(idxv, rows, sem):
            pltpu.make_async_copy(xt_hbm.at[idxv], rows, sem).wait()

        def accum(rows, wgts, cn):
            def q_body(q, inner):
                accs = [jnp.zeros((LANES,), jnp.float32)] * (C // LANES)
                for k in range(4):
                    wv = _bcast(wgts[k], q)
                    m = k * LANES + q
                    for g in range(CW // LANES):
                        wrd = rows[m, pl.ds(g * LANES, LANES)]
                        va, vb = _unpack_words(wrd)
                        accs[g] = accs[g] + wv * va
                        accs[g + 8] = accs[g + 8] + wv * vb
                for g in range(CW // LANES):
                    outbw[q, pl.ds(g * LANES, LANES)] = _pack_words(
                        accs[g], accs[g + 8])
                return inner

            lax.fori_loop(0, CHUNK, q_body, 0)
            pltpu.sync_copy(
                outbw, vw_hbm.at[pl.ds(wid * QPW + cn * CHUNK, CHUNK)])

        pltpu.sync_copy(
            wxy_hbm.at[b, pl.ds(0, 2), pl.ds(wid * QPW, QPW)], gvm)
        wa0 = prep(idxa, 0)
        fire(idxa, rowsa, sema)

        def pair_body(i, wa):
            wb = prep(idxb, 2 * i + 1)
            fire(idxb, rowsb, semb)
            wait(idxa, rowsa, sema)
            accum(rowsa, wa, 2 * i)
            wa2 = prep(idxa, 2 * i + 2)

            @pl.when(i < NCHUNK // 2 - 1)
            def _():
                fire(idxa, rowsa, sema)

            wait(idxb, rowsb, semb)
            accum(rowsb, wb, 2 * i + 1)
            return wa2

        lax.fori_loop(0, NCHUNK // 2, pair_body, wa0)

    return warp_b


_WARPS = [_make_warp(b) for b in range(B)]


BQ = 512


def _proj_body(vw_ref, wc_ref, bc_ref, p_ref):
    j = pl.program_id(0)
    w = vw_ref[...]
    vlow = lax.bitcast_convert_type(lax.shift_left(w, 16), jnp.float32)
    vhigh = lax.bitcast_convert_type(lax.bitwise_and(w, _MASKHI), jnp.float32)
    ot = (lax.dot_general(wc_ref[0:CW], vlow, (((0,), (1,)), ((), ())),
                          preferred_element_type=jnp.float32)
          + lax.dot_general(wc_ref[CW:C], vhigh, (((0,), (1,)), ((), ())),
                            preferred_element_type=jnp.float32))
    ot = ot + bc_ref[...][:, 0:1]
    ot8 = ot[0:8]
    att = ot[8:12]
    m = jnp.max(att, axis=0, keepdims=True)
    e = jnp.exp(att - m)
    aw = e / jnp.sum(e, axis=0, keepdims=True)
    qid = j * BQ + lax.broadcasted_iota(jnp.int32, (8, BQ), 1)
    par = lax.broadcasted_iota(jnp.int32, (8, BQ), 0) & 1
    pxf = (qid & (W - 1)).astype(jnp.float32)
    pyf = (qid >> 7).astype(jnp.float32)
    pos = ot8 + jnp.where(par == 0, pxf, pyf)
    p_ref[0:8, :] = pos
    p_ref[8:12, :] = aw
    p_ref[12:16, :] = jnp.zeros((4, BQ), jnp.float32)


_proj = pl.pallas_call(
    _proj_body,
    grid=(HW // BQ,),
    in_specs=[
        pl.BlockSpec((BQ, CW), lambda j: (j, 0)),
        pl.BlockSpec((C, 32), lambda j: (0, 0)),
        pl.BlockSpec((32, 128), lambda j: (0, 0)),
    ],
    out_specs=pl.BlockSpec((16, BQ), lambda j: (0, j)),
    out_shape=jax.ShapeDtypeStruct((16, HW), jnp.float32),
)


def _make_samp():
    @functools.partial(
        pl.kernel,
        out_type=jax.ShapeDtypeStruct((HW, C), jnp.float32),
        mesh=_mesh,
        scratch_types=[
            pltpu.VMEM((12, QPW), jnp.float32),
            pltpu.VMEM((8 * CHUNK,), jnp.int32),
            pltpu.VMEM((8 * CHUNK,), jnp.int32),
            pltpu.VMEM((8 * CHUNK, CW), jnp.int32),
            pltpu.VMEM((8 * CHUNK, CW), jnp.int32),
            pltpu.VMEM((CHUNK, C), jnp.float32),
            pltpu.SemaphoreType.DMA,
            pltpu.SemaphoreType.DMA,
        ],
        compiler_params=_sc_params,
    )
    def samp_b(v_hbm, p_hbm, o_hbm, pvm, idxa, idxb, bufa, bufb, outb,
               sema, semb):
        wid = lax.axis_index("s") * NC + lax.axis_index("c")

        def prep(p0, idxv, cn):
            off = jnp.minimum(cn, NCHUNK - 1) * CHUNK
            ws = []
            for p in (p0, p0 + 1):
                cw = _corners(pvm[2 * p, pl.ds(off, LANES)],
                              pvm[2 * p + 1, pl.ds(off, LANES)], 0)
                awp = pvm[8 + p, pl.ds(off, LANES)]
                for k, (idx, wgt) in enumerate(cw):
                    r = (p - p0) * 4 + k
                    idxv[pl.ds(r * LANES, LANES)] = idx
                    ws.append(awp * wgt)
            return tuple(ws)

        def fire(idxv, buf, sem):
            pltpu.make_async_copy(v_hbm.at[idxv], buf, sem).start()

        def wait(idxv, buf, sem):
            pltpu.make_async_copy(v_hbm.at[idxv], buf, sem).wait()

        def accum(buf, ws, first):
            def q_body(q, inner):
                if first:
                    accs = [jnp.zeros((LANES,), jnp.float32)] * (C // LANES)
                else:
                    accs = [outb[q, pl.ds(j * LANES, LANES)]
                            for j in range(C // LANES)]
                for r in range(8):
                    wv = _bcast(ws[r], q)
                    m = r * LANES + q
                    for g in range(CW // LANES):
                        wrd = buf[m, pl.ds(g * LANES, LANES)]
                        va, vb = _unpack_words(wrd)
                        accs[g] = accs[g] + wv * va
                        accs[g + 8] = accs[g + 8] + wv * vb
                for j in range(C // LANES):
                    outb[q, pl.ds(j * LANES, LANES)] = accs[j]
                return inner

            lax.fori_loop(0, CHUNK, q_body, 0)

        pltpu.sync_copy(
            p_hbm.at[pl.ds(0, 12), pl.ds(wid * QPW, QPW)], pvm)
        wa0 = prep(0, idxa, 0)
        fire(idxa, bufa, sema)

        def chunk_body(cn, wa):
            wb = prep(2, idxb, cn)
            fire(idxb, bufb, semb)
            wait(idxa, bufa, sema)
            accum(bufa, wa, first=True)
            wa2 = prep(0, idxa, cn + 1)

            @pl.when(cn < NCHUNK - 1)
            def _():
                fire(idxa, bufa, sema)

            wait(idxb, bufb, semb)
            accum(bufb, wb, first=False)
            pltpu.sync_copy(outb, o_hbm.at[pl.ds(wid * QPW + cn * CHUNK, CHUNK)])
            return wa2

        lax.fori_loop(0, NCHUNK, chunk_body, wa0)

    return samp_b


_SAMP = _make_samp()


def kernel(x, record_len, pairwise_t_matrix, W_off, b_off, W_att, b_att):
    del record_len
    xtb = x.reshape(B, C, HW).transpose(0, 2, 1).astype(jnp.bfloat16)
    xtb = xtb.reshape(B * HW, C)
    xtw = lax.bitcast_convert_type(
        jnp.stack([xtb[:, :CW], xtb[:, CW:]], axis=-1),
        jnp.int32)

    theta = pairwise_t_matrix[:, 0, 0].astype(jnp.float32)
    xs = jnp.linspace(-1.0, 1.0, W)
    ys = jnp.linspace(-1.0, 1.0, H)
    gy, gx = jnp.meshgrid(ys, xs, indexing='ij')
    base = jnp.stack([gx, gy, jnp.ones_like(gx)], axis=-1)
    grid = jnp.einsum('nij,hwj->nhwi', theta, base)
    wix = (grid[..., 0].reshape(B, HW) + 1.0) * 0.5 * (W - 1)
    wiy = (grid[..., 1].reshape(B, HW) + 1.0) * 0.5 * (H - 1)
    wxy = jnp.stack([wix, wiy], axis=1)

    wc = jnp.concatenate(
        [W_off[:, :8], W_att[:, :4], jnp.zeros((C, 20), jnp.float32)], axis=1)
    bc = jnp.concatenate(
        [b_off[:8], b_att[:4], jnp.zeros((20,), jnp.float32)])
    bc128 = jnp.broadcast_to(bc[:, None], (32, 128))

    vws = [_WARPS[b](xtw, wxy) for b in range(B)]
    ps = [_proj(vw_b, wc, bc128) for vw_b in vws]
    outs = []
    for b in range(B):
        o_b = _SAMP(vws[b], ps[b])
        outs.append(o_b.T.reshape(1, C, H, W))
    return jnp.concatenate(outs, axis=0)
